# baseline (speedup 1.0000x reference)
import functools

import jax
import jax.numpy as jnp
from jax import lax
from jax.experimental import pallas as pl
from jax.experimental.pallas import tpu as pltpu
from jax.experimental.pallas import tpu_sc as plsc

_BLK = 512
_L = 16


def _gidx(v, idxs):
    dnums = lax.GatherDimensionNumbers(
        offset_dims=(), collapsed_slice_dims=(0,), start_index_map=(0,)
    )
    return lax.gather(
        v, idxs.reshape(_L, 1), dnums, slice_sizes=(1,),
        mode=lax.GatherScatterMode.PROMISE_IN_BOUNDS,
    )


def _bcast(v, lane):
    return _gidx(v, jnp.full((_L,), lane, jnp.int32))


def _butterfly(v, op):
    idx = lax.iota(jnp.int32, _L)
    for k in (1, 2, 4, 8):
        v = op(v, _gidx(v, idx ^ k))
    return v


def _gate_sc_body(sim_hbm, p_hbm, sim_v, p_v):
    pltpu.sync_copy(sim_hbm, sim_v)
    v = sim_v[pl.ds(0, _L)]
    maskv = sim_v[pl.ds(_L, _L)]
    hot0 = sim_v[pl.ds(2 * _L, _L)]
    total = _butterfly(v * maskv, jnp.add)
    s0 = _bcast(v, 0)
    hw = _bcast(v, 8)
    mean = (total - s0) / 7.0
    adj = v * (1.0 - hot0) + mean * hot0
    adjm = adj * maskv + (-1e30) * (1.0 - maskv)
    m = _butterfly(adjm, jnp.maximum)
    ex = jnp.exp(adjm - m) * maskv
    soft = ex / _butterfly(ex, jnp.add)
    onehot = jnp.exp(1e4 * (adjm - m)) * maskv
    p_v[...] = soft + hw * (onehot - soft)
    pltpu.sync_copy(p_v, p_hbm)


_gate_sc = functools.partial(
    pl.kernel,
    out_type=jax.ShapeDtypeStruct((_L,), jnp.float32),
    mesh=plsc.VectorSubcoreMesh(core_axis_name="c", subcore_axis_name="s"),
    scratch_types=[
        pltpu.VMEM((3 * _L,), jnp.float32),
        pltpu.VMEM((_L,), jnp.float32),
    ],
)(_gate_sc_body)


def _scale_body(hw_ref, sim_ref, lm_ref, out_ref, s_ref):
    j = pl.program_id(0)
    per_e = pl.num_programs(0) // sim_ref.shape[1]
    e = j // per_e

    @pl.when(j % per_e == 0)
    def _():
        ne = sim_ref.shape[1]
        sim = sim_ref[...]
        idx = jax.lax.broadcasted_iota(jnp.int32, (1, ne), 1)
        total = jnp.sum(sim)
        s0 = jnp.sum(jnp.where(idx == 0, sim, 0.0))
        mean_rest = (total - s0) / (ne - 1)
        adj = jnp.where(idx == 0, mean_rest, sim)
        m = jnp.max(adj)
        ex = jnp.exp(adj - m)
        soft = ex / jnp.sum(ex)
        amax = jnp.min(jnp.where(adj == m, idx, ne))
        onehot = (idx == amax).astype(jnp.float32)
        pvec = jnp.where(hw_ref[0] > 0.5, onehot, soft)
        s_ref[0] = jnp.sum(jnp.where(idx == e, pvec, 0.0))

    out_ref[...] = lm_ref[...] * s_ref[0]


def kernel(lm_logits, encoder_hidden, decoder_hidden, n_expert, similarity, hard_weight):
    del encoder_hidden, decoder_hidden, n_expert
    ne, Ld, vocab = lm_logits.shape
    simf = similarity.astype(jnp.float32)
    sim2 = simf.reshape(1, ne)
    hw = jnp.asarray(hard_weight, jnp.float32).reshape(1)
    maskv = jnp.concatenate([jnp.ones((8,), jnp.float32), jnp.zeros((8,), jnp.float32)])
    hot0 = jnp.zeros((_L,), jnp.float32).at[0].set(1.0)
    sim48 = jnp.concatenate(
        [simf, hw, jnp.zeros((_L - ne - 1,), jnp.float32), maskv, hot0]
    )
    p16 = _gate_sc(sim48)
    p = jnp.broadcast_to(p16[:ne].reshape(ne, 1, 1), (ne, Ld, 1))
    nblk = ne * Ld // _BLK
    lm_flat = lm_logits.reshape(nblk, _BLK, vocab)
    out = pl.pallas_call(
        _scale_body,
        grid=(nblk,),
        in_specs=[
            pl.BlockSpec(memory_space=pltpu.SMEM),
            pl.BlockSpec((1, ne), lambda j: (0, 0)),
            pl.BlockSpec((1, _BLK, vocab), lambda j: (j, 0, 0)),
        ],
        out_specs=pl.BlockSpec((1, _BLK, vocab), lambda j: (j, 0, 0)),
        out_shape=jax.ShapeDtypeStruct((nblk, _BLK, vocab), jnp.float32),
        scratch_shapes=[pltpu.SMEM((1,), jnp.float32)],
        compiler_params=pltpu.CompilerParams(
            dimension_semantics=("arbitrary",),
        ),
    )(hw, sim2, lm_flat)
    return (out.reshape(ne, Ld, vocab), p)


# speedup vs baseline: 1.0962x; 1.0962x over previous
import jax
import jax.numpy as jnp
from jax.experimental import pallas as pl
from jax.experimental.pallas import tpu as pltpu

_BLK = 512


def _scale_body(hw_ref, sim_ref, lm_ref, out_ref, pw_ref, s_ref):
    j = pl.program_id(0)
    per_e = pl.num_programs(0) // sim_ref.shape[1]
    e = j // per_e

    @pl.when(j % per_e == 0)
    def _():
        ne = sim_ref.shape[1]
        sim = sim_ref[...]
        idx = jax.lax.broadcasted_iota(jnp.int32, (1, ne), 1)
        total = jnp.sum(sim)
        s0 = jnp.sum(jnp.where(idx == 0, sim, 0.0))
        mean_rest = (total - s0) / (ne - 1)
        adj = jnp.where(idx == 0, mean_rest, sim)
        m = jnp.max(adj)
        ex = jnp.exp(adj - m)
        soft = ex / jnp.sum(ex)
        amax = jnp.min(jnp.where(adj == m, idx, ne))
        onehot = (idx == amax).astype(jnp.float32)
        pvec = jnp.where(hw_ref[0] > 0.5, onehot, soft)
        pw_ref[...] = pvec
        s_ref[0] = jnp.sum(jnp.where(idx == e, pvec, 0.0))

    out_ref[...] = lm_ref[...] * s_ref[0]


def kernel(lm_logits, encoder_hidden, decoder_hidden, n_expert, similarity, hard_weight):
    del encoder_hidden, decoder_hidden, n_expert
    ne, Ld, vocab = lm_logits.shape
    sim2 = similarity.astype(jnp.float32).reshape(1, ne)
    hw = jnp.asarray(hard_weight, jnp.float32).reshape(1)
    nblk = ne * Ld // _BLK
    lm_flat = lm_logits.reshape(nblk, _BLK, vocab)
    out, pw = pl.pallas_call(
        _scale_body,
        grid=(nblk,),
        in_specs=[
            pl.BlockSpec(memory_space=pltpu.SMEM),
            pl.BlockSpec((1, ne), lambda j: (0, 0)),
            pl.BlockSpec((1, _BLK, vocab), lambda j: (j, 0, 0)),
        ],
        out_specs=[
            pl.BlockSpec((1, _BLK, vocab), lambda j: (j, 0, 0)),
            pl.BlockSpec((1, ne), lambda j: (0, 0)),
        ],
        out_shape=[
            jax.ShapeDtypeStruct((nblk, _BLK, vocab), jnp.float32),
            jax.ShapeDtypeStruct((1, ne), jnp.float32),
        ],
        scratch_shapes=[pltpu.SMEM((1,), jnp.float32)],
        compiler_params=pltpu.CompilerParams(
            dimension_semantics=("arbitrary",),
        ),
    )(hw, sim2, lm_flat)
    p = jnp.broadcast_to(pw.reshape(ne, 1, 1), (ne, Ld, 1))
    return (out.reshape(ne, Ld, vocab), p)
